# baseline (speedup 1.0000x reference)
import functools

import jax
import jax.numpy as jnp
from jax import lax
from jax.experimental import pallas as pl
from jax.experimental.pallas import tpu as pltpu
from jax.experimental.pallas import tpu_sc as plsc

N = 50000
E = 1600000
G = 64
HID = 128
DC = 16
NCH = 16
K = 128
NI = 14
EPAD = 1605632
NP2 = 50048
NPT = NP2 // 16
BN = 1000
NB = N // BN


def _make_agg(dc, nch, split_edges, nsub):
    nch_per_core = 1 if split_edges else nch // 2
    ept = EPAD // (32 if split_edges else 16)
    trows = ept // K
    brows = NI * nsub
    nblk = trows // brows
    nout = 2 if split_edges else nch
    out_ty = jax.ShapeDtypeStruct((nout * NP2, dc), jnp.float32)
    tab_rows = nch * (N - 1) + 1
    mesh = plsc.VectorSubcoreMesh(core_axis_name="c", subcore_axis_name="s")

    @functools.partial(
        pl.kernel,
        out_type=out_ty,
        mesh=mesh,
        scratch_types=(
            pltpu.VMEM_SHARED((NP2 + 8, dc), jnp.float32),
            pltpu.VMEM((brows, K), jnp.int32),
            pltpu.VMEM((brows, K), jnp.int32),
            pltpu.VMEM((2 * nsub, K, dc), jnp.float32),
            pltpu.SemaphoreType.DMA,
            pltpu.SemaphoreType.DMA,
        ),
        compiler_params=pltpu.CompilerParams(use_tc_tiling_on_sc=False),
    )
    def agg(table_hbm, src_hbm, dst_hbm, zeros_hbm, out_hbm,
            acc, sidx, didx, rows, gsem, ssem):
        cid = lax.axis_index("c")
        sid = lax.axis_index("s")
        for cc in range(nch_per_core):
            if split_edges:
                ch = cid
                tab = table_hbm
                rbase = cid * (EPAD // (2 * K)) + sid * trows
            else:
                ch = cid * nch_per_core + cc
                tab = table_hbm.at[pl.ds(ch, tab_rows)]
                rbase = sid * trows
            pltpu.sync_copy(zeros_hbm, acc.at[pl.ds(sid * NPT, NPT)])
            plsc.subcore_barrier()

            def fire_g(t, sbase):
                return [pltpu.async_copy(tab.at[sidx.at[t * nsub + j]],
                                         rows.at[sbase + j], gsem)
                        for j in range(nsub)]

            def fire_s(t, sbase):
                return [pltpu.async_copy(rows.at[sbase + j],
                                         acc.at[didx.at[t * nsub + j]],
                                         ssem, add=True)
                        for j in range(nsub)]

            def wait_g():
                for j in range(nsub):
                    pltpu.make_async_copy(
                        table_hbm.at[pl.ds(0, K)], rows.at[j], gsem).wait()

            def halfstep(tc, tp, cur):
                if tp is not None:
                    fire_g(tp, nsub - cur)
                wait_g()
                for d in fire_s(tc, cur):
                    d.wait()

            def block(blk, _):
                row0 = rbase + blk * brows
                pltpu.sync_copy(src_hbm.at[pl.ds(row0, brows)], sidx)
                pltpu.sync_copy(dst_hbm.at[pl.ds(row0, brows)], didx)
                fire_g(0, 0)

                def pair(m, _):
                    t = 2 * m
                    halfstep(t, t + 1, 0)
                    halfstep(t + 1, t + 2, nsub)
                    return 0

                lax.fori_loop(0, NI // 2 - 1, pair, 0)
                halfstep(NI - 2, NI - 1, 0)
                halfstep(NI - 1, None, nsub)
                return 0

            lax.fori_loop(0, nblk, block, 0)
            plsc.subcore_barrier()
            pltpu.sync_copy(acc.at[pl.ds(sid * NPT, NPT)],
                            out_hbm.at[pl.ds(ch * NP2 + sid * NPT, NPT)])
            plsc.subcore_barrier()

    return agg


def _layer_body(first):
    def body(h_ref, agg_ref, batch_ref, w1_ref, b1_ref, w2_ref, b2_ref,
             hout_ref, pooled_ref):
        i = pl.program_id(0)
        if first:
            z = h_ref[...] + agg_ref[0] + agg_ref[1]
        else:
            z = h_ref[...] + jnp.concatenate(
                [agg_ref[c] for c in range(NCH)], axis=1)
        y = jnp.maximum(
            jax.lax.dot_general(z, w1_ref[...], (((1,), (0,)), ((), ())),
                                preferred_element_type=jnp.float32)
            + b1_ref[...], 0.0)
        h2 = jnp.maximum(
            jax.lax.dot_general(y, w2_ref[...], (((1,), (0,)), ((), ())),
                                preferred_element_type=jnp.float32)
            + b2_ref[...], 0.0)
        hout_ref[...] = h2
        gids = lax.broadcasted_iota(jnp.int32, (1, G), 1)
        onehot = (batch_ref[...] == gids).astype(jnp.float32)
        p = jax.lax.dot_general(onehot, h2, (((0,), (0,)), ((), ())),
                                preferred_element_type=jnp.float32)

        @pl.when(i == 0)
        def _():
            pooled_ref[...] = p

        @pl.when(i > 0)
        def _():
            pooled_ref[...] = pooled_ref[...] + p

    return body


def _tc_layer(h, agg, batch2d, w1, b1, w2, b2, first):
    if first:
        h_spec = pl.BlockSpec((BN, 8), lambda i: (i, 0))
        agg_spec = pl.BlockSpec((2, BN, 8), lambda i: (0, i, 0))
    else:
        h_spec = pl.BlockSpec((BN, 2 * HID), lambda i: (i, 0))
        agg_spec = pl.BlockSpec((NCH, BN, DC), lambda i: (0, i, 0))
    return pl.pallas_call(
        _layer_body(first),
        grid=(NB,),
        in_specs=[
            h_spec,
            agg_spec,
            pl.BlockSpec((BN, 1), lambda i: (i, 0)),
            pl.BlockSpec(w1.shape, lambda i: (0, 0)),
            pl.BlockSpec(b1.shape, lambda i: (0, 0)),
            pl.BlockSpec(w2.shape, lambda i: (0, 0)),
            pl.BlockSpec(b2.shape, lambda i: (0, 0)),
        ],
        out_specs=[
            pl.BlockSpec((BN, 2 * HID), lambda i: (i, 0)),
            pl.BlockSpec((G, 2 * HID), lambda i: (0, 0)),
        ],
        out_shape=[
            jax.ShapeDtypeStruct((N, 2 * HID), jnp.float32),
            jax.ShapeDtypeStruct((G, 2 * HID), jnp.float32),
        ],
    )(h, agg, batch2d, w1, b1, w2, b2)


def _final_body(p1_ref, p2_ref, p3_ref, w1_ref, b1_ref, w2_ref, b2_ref,
                out_ref):
    parts = []
    for p in (p1_ref, p2_ref, p3_ref):
        v = p[...]
        parts.append(0.5 * (v[:, :HID] + v[:, HID:]))
    h = jnp.concatenate(parts, axis=1)
    y = jnp.maximum(
        jax.lax.dot_general(h, w1_ref[...], (((1,), (0,)), ((), ())),
                            preferred_element_type=jnp.float32)
        + b1_ref[...], 0.0)
    out_ref[...] = jax.lax.dot_general(
        y, w2_ref[...], (((1,), (0,)), ((), ())),
        preferred_element_type=jnp.float32) + b2_ref[...]


def _blockdiag(a, b):
    z = jnp.zeros((a.shape[0] + b.shape[0], a.shape[1] + b.shape[1]),
                  jnp.float32)
    z = z.at[:a.shape[0], :a.shape[1]].set(a)
    return z.at[a.shape[0]:, a.shape[1]:].set(b)


def kernel(pos, x, edge_index, batch, geo_params, topo_params, emb_params):
    f32 = jnp.float32
    src = edge_index[0].astype(jnp.int32)
    dst = edge_index[1].astype(jnp.int32)
    npad = EPAD - E
    src_p = jnp.concatenate([src, jnp.zeros((npad,), jnp.int32)])
    src16_p = (src_p * NCH).reshape(EPAD // K, K)
    src_p = src_p.reshape(EPAD // K, K)
    dst_p = jnp.concatenate([dst, jnp.full((npad,), NP2, jnp.int32)])
    dst_p = dst_p.reshape(EPAD // K, K)
    feat0 = jnp.concatenate([pos.astype(f32), x[:, 3:8].astype(f32)], axis=1)
    batch2d = batch.astype(jnp.int32).reshape(N, 1)
    zeros0 = jnp.zeros((NPT, 8), f32)
    zeros1 = jnp.zeros((NPT, DC), f32)

    w1c, b1c, w2c, b2c = [], [], [], []
    for li in range(3):
        (w1g, b1g), (w2g, b2g) = geo_params[li]
        (w1t, b1t), (w2t, b2t) = topo_params[li]
        w1c.append(_blockdiag(w1g, w1t))
        b1c.append(jnp.concatenate([b1g, b1t]).reshape(1, 2 * HID))
        w2c.append(_blockdiag(w2g, w2t))
        b2c.append(jnp.concatenate([b2g, b2t]).reshape(1, 2 * HID))
    (we1, be1), (we2, be2) = emb_params
    be1 = be1.reshape(1, -1)
    be2 = be2.reshape(1, -1)

    agg0_fn = _make_agg(8, 1, True, 4)
    agg_fn = _make_agg(DC, NCH, False, 8)

    agg0 = agg0_fn(feat0, src_p, dst_p, zeros0).reshape(2, NP2, 8)
    h1, p1 = _tc_layer(feat0, agg0, batch2d,
                       w1c[0], b1c[0], w2c[0], b2c[0], first=True)

    agg1 = agg_fn(h1.reshape(NCH * N, DC), src16_p, dst_p,
                  zeros1).reshape(NCH, NP2, DC)
    h2, p2 = _tc_layer(h1, agg1, batch2d,
                       w1c[1], b1c[1], w2c[1], b2c[1], first=False)

    agg2 = agg_fn(h2.reshape(NCH * N, DC), src16_p, dst_p,
                  zeros1).reshape(NCH, NP2, DC)
    h3, p3 = _tc_layer(h2, agg2, batch2d,
                       w1c[2], b1c[2], w2c[2], b2c[2], first=False)

    out = pl.pallas_call(
        _final_body,
        out_shape=jax.ShapeDtypeStruct((G, we2.shape[1]), f32),
    )(p1, p2, p3, we1, be1, we2, be2)
    return out


# speedup vs baseline: 1.1471x; 1.1471x over previous
import functools

import jax
import jax.numpy as jnp
from jax import lax
from jax.experimental import pallas as pl
from jax.experimental.pallas import tpu as pltpu
from jax.experimental.pallas import tpu_sc as plsc

N = 50000
E = 1600000
G = 64
HID = 128
DC = 32
NCH = 8
K = 128
NI = 14
EPAD = 1605632
NP2 = 50048
NPT = NP2 // 16
BN = 1000
NB = N // BN


def _make_agg(dc, nch, split_edges, nsub):
    nch_per_core = 1 if split_edges else nch // 2
    ept = EPAD // (32 if split_edges else 16)
    trows = ept // K
    brows = NI * nsub
    nblk = trows // brows
    nout = 2 if split_edges else nch
    out_ty = jax.ShapeDtypeStruct((nout * NP2, dc), jnp.float32)
    tab_rows = nch * (N - 1) + 1
    mesh = plsc.VectorSubcoreMesh(core_axis_name="c", subcore_axis_name="s")

    @functools.partial(
        pl.kernel,
        out_type=out_ty,
        mesh=mesh,
        scratch_types=(
            pltpu.VMEM_SHARED((NP2 + 8, dc), jnp.float32),
            pltpu.VMEM((brows, K), jnp.int32),
            pltpu.VMEM((brows, K), jnp.int32),
            pltpu.VMEM((2 * nsub, K, dc), jnp.float32),
            pltpu.SemaphoreType.DMA,
            pltpu.SemaphoreType.DMA,
        ),
        compiler_params=pltpu.CompilerParams(use_tc_tiling_on_sc=False),
    )
    def agg(table_hbm, src_hbm, dst_hbm, zeros_hbm, out_hbm,
            acc, sidx, didx, rows, gsem, ssem):
        cid = lax.axis_index("c")
        sid = lax.axis_index("s")
        for cc in range(nch_per_core):
            if split_edges:
                ch = cid
                tab = table_hbm
                rbase = cid * (EPAD // (2 * K)) + sid * trows
            else:
                ch = cid * nch_per_core + cc
                tab = table_hbm.at[pl.ds(ch, tab_rows)]
                rbase = sid * trows
            pltpu.sync_copy(zeros_hbm, acc.at[pl.ds(sid * NPT, NPT)])
            plsc.subcore_barrier()

            def fire_g(t, sbase):
                return [pltpu.async_copy(tab.at[sidx.at[t * nsub + j]],
                                         rows.at[sbase + j], gsem)
                        for j in range(nsub)]

            def fire_s(t, sbase):
                return [pltpu.async_copy(rows.at[sbase + j],
                                         acc.at[didx.at[t * nsub + j]],
                                         ssem, add=True)
                        for j in range(nsub)]

            def wait_g():
                for j in range(nsub):
                    pltpu.make_async_copy(
                        table_hbm.at[pl.ds(0, K)], rows.at[j], gsem).wait()

            def halfstep(tc, tp, cur):
                if tp is not None:
                    fire_g(tp, nsub - cur)
                wait_g()
                for d in fire_s(tc, cur):
                    d.wait()

            def block(blk, _):
                row0 = rbase + blk * brows
                pltpu.sync_copy(src_hbm.at[pl.ds(row0, brows)], sidx)
                pltpu.sync_copy(dst_hbm.at[pl.ds(row0, brows)], didx)
                fire_g(0, 0)

                def pair(m, _):
                    t = 2 * m
                    halfstep(t, t + 1, 0)
                    halfstep(t + 1, t + 2, nsub)
                    return 0

                lax.fori_loop(0, NI // 2 - 1, pair, 0)
                halfstep(NI - 2, NI - 1, 0)
                halfstep(NI - 1, None, nsub)
                return 0

            lax.fori_loop(0, nblk, block, 0)
            plsc.subcore_barrier()
            pltpu.sync_copy(acc.at[pl.ds(sid * NPT, NPT)],
                            out_hbm.at[pl.ds(ch * NP2 + sid * NPT, NPT)])
            plsc.subcore_barrier()

    return agg


def _layer_body(first):
    def body(h_ref, agg_ref, batch_ref, w1_ref, b1_ref, w2_ref, b2_ref,
             hout_ref, pooled_ref):
        i = pl.program_id(0)
        if first:
            z = h_ref[...] + agg_ref[0] + agg_ref[1]
        else:
            z = h_ref[...] + jnp.concatenate(
                [agg_ref[c] for c in range(NCH)], axis=1)
        y = jnp.maximum(
            jax.lax.dot_general(z, w1_ref[...], (((1,), (0,)), ((), ())),
                                preferred_element_type=jnp.float32)
            + b1_ref[...], 0.0)
        h2 = jnp.maximum(
            jax.lax.dot_general(y, w2_ref[...], (((1,), (0,)), ((), ())),
                                preferred_element_type=jnp.float32)
            + b2_ref[...], 0.0)
        hout_ref[...] = h2
        gids = lax.broadcasted_iota(jnp.int32, (1, G), 1)
        onehot = (batch_ref[...] == gids).astype(jnp.float32)
        p = jax.lax.dot_general(onehot, h2, (((0,), (0,)), ((), ())),
                                preferred_element_type=jnp.float32)

        @pl.when(i == 0)
        def _():
            pooled_ref[...] = p

        @pl.when(i > 0)
        def _():
            pooled_ref[...] = pooled_ref[...] + p

    return body


def _tc_layer(h, agg, batch2d, w1, b1, w2, b2, first):
    if first:
        h_spec = pl.BlockSpec((BN, 8), lambda i: (i, 0))
        agg_spec = pl.BlockSpec((2, BN, 8), lambda i: (0, i, 0))
    else:
        h_spec = pl.BlockSpec((BN, 2 * HID), lambda i: (i, 0))
        agg_spec = pl.BlockSpec((NCH, BN, DC), lambda i: (0, i, 0))
    return pl.pallas_call(
        _layer_body(first),
        grid=(NB,),
        in_specs=[
            h_spec,
            agg_spec,
            pl.BlockSpec((BN, 1), lambda i: (i, 0)),
            pl.BlockSpec(w1.shape, lambda i: (0, 0)),
            pl.BlockSpec(b1.shape, lambda i: (0, 0)),
            pl.BlockSpec(w2.shape, lambda i: (0, 0)),
            pl.BlockSpec(b2.shape, lambda i: (0, 0)),
        ],
        out_specs=[
            pl.BlockSpec((BN, 2 * HID), lambda i: (i, 0)),
            pl.BlockSpec((G, 2 * HID), lambda i: (0, 0)),
        ],
        out_shape=[
            jax.ShapeDtypeStruct((N, 2 * HID), jnp.float32),
            jax.ShapeDtypeStruct((G, 2 * HID), jnp.float32),
        ],
    )(h, agg, batch2d, w1, b1, w2, b2)


def _final_body(p1_ref, p2_ref, p3_ref, w1_ref, b1_ref, w2_ref, b2_ref,
                out_ref):
    parts = []
    for p in (p1_ref, p2_ref, p3_ref):
        v = p[...]
        parts.append(0.5 * (v[:, :HID] + v[:, HID:]))
    h = jnp.concatenate(parts, axis=1)
    y = jnp.maximum(
        jax.lax.dot_general(h, w1_ref[...], (((1,), (0,)), ((), ())),
                            preferred_element_type=jnp.float32)
        + b1_ref[...], 0.0)
    out_ref[...] = jax.lax.dot_general(
        y, w2_ref[...], (((1,), (0,)), ((), ())),
        preferred_element_type=jnp.float32) + b2_ref[...]


def _blockdiag(a, b):
    z = jnp.zeros((a.shape[0] + b.shape[0], a.shape[1] + b.shape[1]),
                  jnp.float32)
    z = z.at[:a.shape[0], :a.shape[1]].set(a)
    return z.at[a.shape[0]:, a.shape[1]:].set(b)


def kernel(pos, x, edge_index, batch, geo_params, topo_params, emb_params):
    f32 = jnp.float32
    src = edge_index[0].astype(jnp.int32)
    dst = edge_index[1].astype(jnp.int32)
    npad = EPAD - E
    src_p = jnp.concatenate([src, jnp.zeros((npad,), jnp.int32)])
    src16_p = (src_p * NCH).reshape(EPAD // K, K)
    src_p = src_p.reshape(EPAD // K, K)
    dst_p = jnp.concatenate([dst, jnp.full((npad,), NP2, jnp.int32)])
    dst_p = dst_p.reshape(EPAD // K, K)
    feat0 = jnp.concatenate([pos.astype(f32), x[:, 3:8].astype(f32)], axis=1)
    batch2d = batch.astype(jnp.int32).reshape(N, 1)
    zeros0 = jnp.zeros((NPT, 8), f32)
    zeros1 = jnp.zeros((NPT, DC), f32)

    w1c, b1c, w2c, b2c = [], [], [], []
    for li in range(3):
        (w1g, b1g), (w2g, b2g) = geo_params[li]
        (w1t, b1t), (w2t, b2t) = topo_params[li]
        w1c.append(_blockdiag(w1g, w1t))
        b1c.append(jnp.concatenate([b1g, b1t]).reshape(1, 2 * HID))
        w2c.append(_blockdiag(w2g, w2t))
        b2c.append(jnp.concatenate([b2g, b2t]).reshape(1, 2 * HID))
    (we1, be1), (we2, be2) = emb_params
    be1 = be1.reshape(1, -1)
    be2 = be2.reshape(1, -1)

    agg0_fn = _make_agg(8, 1, True, 4)
    agg_fn = _make_agg(DC, NCH, False, 2)

    agg0 = agg0_fn(feat0, src_p, dst_p, zeros0).reshape(2, NP2, 8)
    h1, p1 = _tc_layer(feat0, agg0, batch2d,
                       w1c[0], b1c[0], w2c[0], b2c[0], first=True)

    agg1 = agg_fn(h1.reshape(NCH * N, DC), src16_p, dst_p,
                  zeros1).reshape(NCH, NP2, DC)
    h2, p2 = _tc_layer(h1, agg1, batch2d,
                       w1c[1], b1c[1], w2c[1], b2c[1], first=False)

    agg2 = agg_fn(h2.reshape(NCH * N, DC), src16_p, dst_p,
                  zeros1).reshape(NCH, NP2, DC)
    h3, p3 = _tc_layer(h2, agg2, batch2d,
                       w1c[2], b1c[2], w2c[2], b2c[2], first=False)

    out = pl.pallas_call(
        _final_body,
        out_shape=jax.ShapeDtypeStruct((G, we2.shape[1]), f32),
    )(p1, p2, p3, we1, be1, we2, be2)
    return out


# speedup vs baseline: 1.1557x; 1.0074x over previous
import functools

import jax
import jax.numpy as jnp
from jax import lax
from jax.experimental import pallas as pl
from jax.experimental.pallas import tpu as pltpu
from jax.experimental.pallas import tpu_sc as plsc

N = 50000
E = 1600000
G = 64
HID = 128
DC = 32
NCH = 8
K = 128
NI = 14
NI2 = 28
EPAD = 1605632
NP2 = 50048
NPT = NP2 // 16
BN = 1000
NB = N // BN


def _make_agg(dc, nch, split_edges, nsub, ni):
    nch_per_core = 1 if split_edges else nch // 2
    ept = EPAD // (32 if split_edges else 16)
    trows = ept // K
    brows = ni * nsub
    nblk = trows // brows
    nout = 2 if split_edges else nch
    out_ty = jax.ShapeDtypeStruct((nout * NP2, dc), jnp.float32)
    tab_rows = nch * (N - 1) + 1
    mesh = plsc.VectorSubcoreMesh(core_axis_name="c", subcore_axis_name="s")

    @functools.partial(
        pl.kernel,
        out_type=out_ty,
        mesh=mesh,
        scratch_types=(
            pltpu.VMEM_SHARED((NP2 + 8, dc), jnp.float32),
            pltpu.VMEM((brows, K), jnp.int32),
            pltpu.VMEM((brows, K), jnp.int32),
            pltpu.VMEM((2 * nsub, K, dc), jnp.float32),
            pltpu.SemaphoreType.DMA,
            pltpu.SemaphoreType.DMA,
        ),
        compiler_params=pltpu.CompilerParams(use_tc_tiling_on_sc=False),
    )
    def agg(table_hbm, src_hbm, dst_hbm, zeros_hbm, out_hbm,
            acc, sidx, didx, rows, gsem, ssem):
        cid = lax.axis_index("c")
        sid = lax.axis_index("s")
        for cc in range(nch_per_core):
            if split_edges:
                ch = cid
                tab = table_hbm
                rbase = cid * (EPAD // (2 * K)) + sid * trows
            else:
                ch = cid * nch_per_core + cc
                tab = table_hbm.at[pl.ds(ch, tab_rows)]
                rbase = sid * trows
            pltpu.sync_copy(zeros_hbm, acc.at[pl.ds(sid * NPT, NPT)])
            plsc.subcore_barrier()

            def fire_g(t, sbase):
                return [pltpu.async_copy(tab.at[sidx.at[t * nsub + j]],
                                         rows.at[sbase + j], gsem)
                        for j in range(nsub)]

            def fire_s(t, sbase):
                return [pltpu.async_copy(rows.at[sbase + j],
                                         acc.at[didx.at[t * nsub + j]],
                                         ssem, add=True)
                        for j in range(nsub)]

            def wait_g():
                for j in range(nsub):
                    pltpu.make_async_copy(
                        table_hbm.at[pl.ds(0, K)], rows.at[j], gsem).wait()

            def halfstep(tc, tp, cur):
                if tp is not None:
                    fire_g(tp, nsub - cur)
                wait_g()
                for d in fire_s(tc, cur):
                    d.wait()

            def block(blk, _):
                row0 = rbase + blk * brows
                pltpu.sync_copy(src_hbm.at[pl.ds(row0, brows)], sidx)
                pltpu.sync_copy(dst_hbm.at[pl.ds(row0, brows)], didx)
                if not split_edges:
                    for r in range(brows):
                        for q in range(K // 16):
                            sl = pl.ds(q * 16, 16)
                            sidx[r, sl] = sidx[r, sl] * nch
                fire_g(0, 0)

                def pair(m, _):
                    t = 2 * m
                    halfstep(t, t + 1, 0)
                    halfstep(t + 1, t + 2, nsub)
                    return 0

                lax.fori_loop(0, ni // 2 - 1, pair, 0)
                halfstep(ni - 2, ni - 1, 0)
                halfstep(ni - 1, None, nsub)
                return 0

            lax.fori_loop(0, nblk, block, 0)
            plsc.subcore_barrier()
            pltpu.sync_copy(acc.at[pl.ds(sid * NPT, NPT)],
                            out_hbm.at[pl.ds(ch * NP2 + sid * NPT, NPT)])
            plsc.subcore_barrier()

    return agg


def _layer_body(first):
    def body(h_ref, agg_ref, batch_ref, w1_ref, b1_ref, w2_ref, b2_ref,
             hout_ref, pooled_ref):
        i = pl.program_id(0)
        if first:
            z = h_ref[...] + agg_ref[0] + agg_ref[1]
        else:
            z = h_ref[...] + jnp.concatenate(
                [agg_ref[c] for c in range(NCH)], axis=1)
        y = jnp.maximum(
            jax.lax.dot_general(z, w1_ref[...], (((1,), (0,)), ((), ())),
                                preferred_element_type=jnp.float32)
            + b1_ref[...], 0.0)
        h2 = jnp.maximum(
            jax.lax.dot_general(y, w2_ref[...], (((1,), (0,)), ((), ())),
                                preferred_element_type=jnp.float32)
            + b2_ref[...], 0.0)
        hout_ref[...] = h2
        gids = lax.broadcasted_iota(jnp.int32, (1, G), 1)
        onehot = (batch_ref[...] == gids).astype(jnp.float32)
        p = jax.lax.dot_general(onehot, h2, (((0,), (0,)), ((), ())),
                                preferred_element_type=jnp.float32)

        @pl.when(i == 0)
        def _():
            pooled_ref[...] = p

        @pl.when(i > 0)
        def _():
            pooled_ref[...] = pooled_ref[...] + p

    return body


def _tc_layer(h, agg, batch2d, w1, b1, w2, b2, first):
    if first:
        h_spec = pl.BlockSpec((BN, 8), lambda i: (i, 0))
        agg_spec = pl.BlockSpec((2, BN, 8), lambda i: (0, i, 0))
    else:
        h_spec = pl.BlockSpec((BN, 2 * HID), lambda i: (i, 0))
        agg_spec = pl.BlockSpec((NCH, BN, DC), lambda i: (0, i, 0))
    return pl.pallas_call(
        _layer_body(first),
        grid=(NB,),
        in_specs=[
            h_spec,
            agg_spec,
            pl.BlockSpec((BN, 1), lambda i: (i, 0)),
            pl.BlockSpec(w1.shape, lambda i: (0, 0)),
            pl.BlockSpec(b1.shape, lambda i: (0, 0)),
            pl.BlockSpec(w2.shape, lambda i: (0, 0)),
            pl.BlockSpec(b2.shape, lambda i: (0, 0)),
        ],
        out_specs=[
            pl.BlockSpec((BN, 2 * HID), lambda i: (i, 0)),
            pl.BlockSpec((G, 2 * HID), lambda i: (0, 0)),
        ],
        out_shape=[
            jax.ShapeDtypeStruct((N, 2 * HID), jnp.float32),
            jax.ShapeDtypeStruct((G, 2 * HID), jnp.float32),
        ],
    )(h, agg, batch2d, w1, b1, w2, b2)


def _last_body(h_ref, agg_ref, batch_ref, w1_ref, b1_ref, w2_ref, b2_ref,
               p1_ref, p2_ref, we1_ref, be1_ref, we2_ref, be2_ref,
               out_ref, pool_acc):
    i = pl.program_id(0)
    z = h_ref[...] + jnp.concatenate(
        [agg_ref[c] for c in range(NCH)], axis=1)
    y = jnp.maximum(
        jax.lax.dot_general(z, w1_ref[...], (((1,), (0,)), ((), ())),
                            preferred_element_type=jnp.float32)
        + b1_ref[...], 0.0)
    h2 = jnp.maximum(
        jax.lax.dot_general(y, w2_ref[...], (((1,), (0,)), ((), ())),
                            preferred_element_type=jnp.float32)
        + b2_ref[...], 0.0)
    gids = lax.broadcasted_iota(jnp.int32, (1, G), 1)
    onehot = (batch_ref[...] == gids).astype(jnp.float32)
    p = jax.lax.dot_general(onehot, h2, (((0,), (0,)), ((), ())),
                            preferred_element_type=jnp.float32)

    @pl.when(i == 0)
    def _():
        pool_acc[...] = p

    @pl.when(i > 0)
    def _():
        pool_acc[...] = pool_acc[...] + p

    @pl.when(i == NB - 1)
    def _():
        parts = []
        for v in (p1_ref[...], p2_ref[...], pool_acc[...]):
            parts.append(0.5 * (v[:, :HID] + v[:, HID:]))
        hcat = jnp.concatenate(parts, axis=1)
        t = jnp.maximum(
            jax.lax.dot_general(hcat, we1_ref[...], (((1,), (0,)), ((), ())),
                                preferred_element_type=jnp.float32)
            + be1_ref[...], 0.0)
        out_ref[...] = jax.lax.dot_general(
            t, we2_ref[...], (((1,), (0,)), ((), ())),
            preferred_element_type=jnp.float32) + be2_ref[...]


def _tc_last(h, agg, batch2d, w1, b1, w2, b2, p1, p2, we1, be1, we2, be2):
    return pl.pallas_call(
        _last_body,
        grid=(NB,),
        in_specs=[
            pl.BlockSpec((BN, 2 * HID), lambda i: (i, 0)),
            pl.BlockSpec((NCH, BN, DC), lambda i: (0, i, 0)),
            pl.BlockSpec((BN, 1), lambda i: (i, 0)),
            pl.BlockSpec(w1.shape, lambda i: (0, 0)),
            pl.BlockSpec(b1.shape, lambda i: (0, 0)),
            pl.BlockSpec(w2.shape, lambda i: (0, 0)),
            pl.BlockSpec(b2.shape, lambda i: (0, 0)),
            pl.BlockSpec(p1.shape, lambda i: (0, 0)),
            pl.BlockSpec(p2.shape, lambda i: (0, 0)),
            pl.BlockSpec(we1.shape, lambda i: (0, 0)),
            pl.BlockSpec(be1.shape, lambda i: (0, 0)),
            pl.BlockSpec(we2.shape, lambda i: (0, 0)),
            pl.BlockSpec(be2.shape, lambda i: (0, 0)),
        ],
        out_specs=pl.BlockSpec((G, we2.shape[1]), lambda i: (0, 0)),
        out_shape=jax.ShapeDtypeStruct((G, we2.shape[1]), jnp.float32),
        scratch_shapes=[pltpu.VMEM((G, 2 * HID), jnp.float32)],
    )(h, agg, batch2d, w1, b1, w2, b2, p1, p2, we1, be1, we2, be2)


def _blockdiag(a, b):
    z = jnp.zeros((a.shape[0] + b.shape[0], a.shape[1] + b.shape[1]),
                  jnp.float32)
    z = z.at[:a.shape[0], :a.shape[1]].set(a)
    return z.at[a.shape[0]:, a.shape[1]:].set(b)


def kernel(pos, x, edge_index, batch, geo_params, topo_params, emb_params):
    f32 = jnp.float32
    src = edge_index[0].astype(jnp.int32)
    dst = edge_index[1].astype(jnp.int32)
    npad = EPAD - E
    src_p = jnp.concatenate([src, jnp.zeros((npad,), jnp.int32)])
    src_p = src_p.reshape(EPAD // K, K)
    dst_p = jnp.concatenate([dst, jnp.full((npad,), NP2, jnp.int32)])
    dst_p = dst_p.reshape(EPAD // K, K)
    feat0 = jnp.concatenate([pos.astype(f32), x[:, 3:8].astype(f32)], axis=1)
    batch2d = batch.astype(jnp.int32).reshape(N, 1)
    zeros0 = jnp.zeros((NPT, 8), f32)
    zeros1 = jnp.zeros((NPT, DC), f32)

    w1c, b1c, w2c, b2c = [], [], [], []
    for li in range(3):
        (w1g, b1g), (w2g, b2g) = geo_params[li]
        (w1t, b1t), (w2t, b2t) = topo_params[li]
        w1c.append(_blockdiag(w1g, w1t))
        b1c.append(jnp.concatenate([b1g, b1t]).reshape(1, 2 * HID))
        w2c.append(_blockdiag(w2g, w2t))
        b2c.append(jnp.concatenate([b2g, b2t]).reshape(1, 2 * HID))
    (we1, be1), (we2, be2) = emb_params
    be1 = be1.reshape(1, -1)
    be2 = be2.reshape(1, -1)

    agg0_fn = _make_agg(8, 1, True, 4, NI)
    agg_fn = _make_agg(DC, NCH, False, 2, NI2)

    agg0 = agg0_fn(feat0, src_p, dst_p, zeros0).reshape(2, NP2, 8)
    h1, p1 = _tc_layer(feat0, agg0, batch2d,
                       w1c[0], b1c[0], w2c[0], b2c[0], first=True)

    agg1 = agg_fn(h1.reshape(NCH * N, DC), src_p, dst_p,
                  zeros1).reshape(NCH, NP2, DC)
    h2, p2 = _tc_layer(h1, agg1, batch2d,
                       w1c[1], b1c[1], w2c[1], b2c[1], first=False)

    agg2 = agg_fn(h2.reshape(NCH * N, DC), src_p, dst_p,
                  zeros1).reshape(NCH, NP2, DC)
    out = _tc_last(h2, agg2, batch2d, w1c[2], b1c[2], w2c[2], b2c[2],
                   p1, p2, we1, be1, we2, be2)
    return out


# speedup vs baseline: 1.1793x; 1.0204x over previous
import functools

import jax
import jax.numpy as jnp
from jax import lax
from jax.experimental import pallas as pl
from jax.experimental.pallas import tpu as pltpu
from jax.experimental.pallas import tpu_sc as plsc

N = 50000
E = 1600000
G = 64
HID = 128
DC = 32
NCH = 8
K = 128
NI = 14
NI2 = 28
EPAD = 1605632
NP2 = 50048
NPT = NP2 // 16
BN = 1000
NB = N // BN


def _make_agg(dc, nch, split_edges, nsub, ni):
    nch_per_core = 1 if split_edges else nch // 2
    ept = EPAD // (32 if split_edges else 16)
    trows = ept // K
    brows = ni * nsub
    nblk = trows // brows
    nout = 2 if split_edges else nch
    out_ty = jax.ShapeDtypeStruct((nout * NP2, dc), jnp.float32)
    tab_rows = nch * (N - 1) + 1
    mesh = plsc.VectorSubcoreMesh(core_axis_name="c", subcore_axis_name="s")

    @functools.partial(
        pl.kernel,
        out_type=out_ty,
        mesh=mesh,
        scratch_types=(
            pltpu.VMEM_SHARED((NP2 + 8, dc), jnp.float32),
            pltpu.VMEM((brows, K), jnp.int32),
            pltpu.VMEM((brows, K), jnp.int32),
            pltpu.VMEM((2 * nsub, K, dc), jnp.float32),
            pltpu.SemaphoreType.DMA,
            pltpu.SemaphoreType.DMA,
        ),
        compiler_params=pltpu.CompilerParams(use_tc_tiling_on_sc=False),
    )
    def agg(table_hbm, src_hbm, dst_hbm, zeros_hbm, out_hbm,
            acc, sidx, didx, rows, gsem, ssem):
        cid = lax.axis_index("c")
        sid = lax.axis_index("s")
        for cc in range(nch_per_core):
            if split_edges:
                ch = cid
                tab = table_hbm
                rbase = cid * (EPAD // (2 * K)) + sid * trows
            else:
                ch = cid * nch_per_core + cc
                tab = table_hbm.at[pl.ds(ch, tab_rows)]
                rbase = sid * trows
            pltpu.sync_copy(zeros_hbm, acc.at[pl.ds(sid * NPT, NPT)])
            plsc.subcore_barrier()

            def fire_g(t, sbase):
                return [pltpu.async_copy(tab.at[sidx.at[t * nsub + j]],
                                         rows.at[sbase + j], gsem)
                        for j in range(nsub)]

            def fire_s(t, sbase):
                return [pltpu.async_copy(rows.at[sbase + j],
                                         acc.at[didx.at[t * nsub + j]],
                                         ssem, add=True)
                        for j in range(nsub)]

            def wait_g():
                for j in range(nsub):
                    pltpu.make_async_copy(
                        table_hbm.at[pl.ds(0, K)], rows.at[j], gsem).wait()

            def halfstep(tc, tp, cur):
                if tp is not None:
                    fire_g(tp, nsub - cur)
                wait_g()
                for d in fire_s(tc, cur):
                    d.wait()

            def block(blk, _):
                row0 = rbase + blk * brows
                pltpu.sync_copy(src_hbm.at[pl.ds(row0, brows)], sidx)
                pltpu.sync_copy(dst_hbm.at[pl.ds(row0, brows)], didx)
                if not split_edges:
                    for r in range(brows):
                        for q in range(K // 16):
                            sl = pl.ds(q * 16, 16)
                            sidx[r, sl] = sidx[r, sl] * nch
                fire_g(0, 0)

                def pair(m, _):
                    t = 2 * m
                    halfstep(t, t + 1, 0)
                    halfstep(t + 1, t + 2, nsub)
                    return 0

                lax.fori_loop(0, ni // 2 - 1, pair, 0)
                halfstep(ni - 2, ni - 1, 0)
                halfstep(ni - 1, None, nsub)
                return 0

            lax.fori_loop(0, nblk, block, 0)
            plsc.subcore_barrier()
            pltpu.sync_copy(acc.at[pl.ds(sid * NPT, NPT)],
                            out_hbm.at[pl.ds(ch * NP2 + sid * NPT, NPT)])
            plsc.subcore_barrier()

    return agg


def _layer_body(first):
    def body(h_ref, agg_ref, batch_ref, w1_ref, b1_ref, w2_ref, b2_ref,
             hout_ref, pooled_ref):
        i = pl.program_id(0)
        if first:
            z = h_ref[...] + agg_ref[0] + agg_ref[1]
        else:
            z = h_ref[...] + jnp.concatenate(
                [agg_ref[c] for c in range(NCH)], axis=1)
        y = jnp.maximum(
            jax.lax.dot_general(z, w1_ref[...], (((1,), (0,)), ((), ())),
                                preferred_element_type=jnp.float32)
            + b1_ref[...], 0.0)
        h2 = jnp.maximum(
            jax.lax.dot_general(y, w2_ref[...], (((1,), (0,)), ((), ())),
                                preferred_element_type=jnp.float32)
            + b2_ref[...], 0.0)
        hout_ref[...] = h2
        gids = lax.broadcasted_iota(jnp.int32, (1, G), 1)
        onehot = (batch_ref[...] == gids).astype(jnp.float32)
        p = jax.lax.dot_general(onehot, h2, (((0,), (0,)), ((), ())),
                                preferred_element_type=jnp.float32)

        @pl.when(i == 0)
        def _():
            pooled_ref[...] = p

        @pl.when(i > 0)
        def _():
            pooled_ref[...] = pooled_ref[...] + p

    return body


def _tc_layer(h, agg, batch2d, w1, b1, w2, b2, first):
    if first:
        h_spec = pl.BlockSpec((BN, 8), lambda i: (i, 0))
        agg_spec = pl.BlockSpec((2, BN, 8), lambda i: (0, i, 0))
    else:
        h_spec = pl.BlockSpec((BN, 2 * HID), lambda i: (i, 0))
        agg_spec = pl.BlockSpec((NCH, BN, DC), lambda i: (0, i, 0))
    return pl.pallas_call(
        _layer_body(first),
        grid=(NB,),
        in_specs=[
            h_spec,
            agg_spec,
            pl.BlockSpec((BN, 1), lambda i: (i, 0)),
            pl.BlockSpec(w1.shape, lambda i: (0, 0)),
            pl.BlockSpec(b1.shape, lambda i: (0, 0)),
            pl.BlockSpec(w2.shape, lambda i: (0, 0)),
            pl.BlockSpec(b2.shape, lambda i: (0, 0)),
        ],
        out_specs=[
            pl.BlockSpec((BN, 2 * HID), lambda i: (i, 0)),
            pl.BlockSpec((G, 2 * HID), lambda i: (0, 0)),
        ],
        out_shape=[
            jax.ShapeDtypeStruct((N, 2 * HID), jnp.float32),
            jax.ShapeDtypeStruct((G, 2 * HID), jnp.float32),
        ],
    )(h, agg, batch2d, w1, b1, w2, b2)


def _last_body(h_ref, agg_ref, batch_ref, w1_ref, b1_ref, w2_ref, b2_ref,
               p1_ref, p2_ref, we1_ref, be1_ref, we2_ref, be2_ref,
               out_ref, pool_acc):
    i = pl.program_id(0)
    z = h_ref[...] + jnp.concatenate(
        [agg_ref[c] for c in range(NCH)], axis=1)
    y = jnp.maximum(
        jax.lax.dot_general(z, w1_ref[...], (((1,), (0,)), ((), ())),
                            preferred_element_type=jnp.float32)
        + b1_ref[...], 0.0)
    h2 = jnp.maximum(
        jax.lax.dot_general(y, w2_ref[...], (((1,), (0,)), ((), ())),
                            preferred_element_type=jnp.float32)
        + b2_ref[...], 0.0)
    gids = lax.broadcasted_iota(jnp.int32, (1, G), 1)
    onehot = (batch_ref[...] == gids).astype(jnp.float32)
    p = jax.lax.dot_general(onehot, h2, (((0,), (0,)), ((), ())),
                            preferred_element_type=jnp.float32)

    @pl.when(i == 0)
    def _():
        pool_acc[...] = p

    @pl.when(i > 0)
    def _():
        pool_acc[...] = pool_acc[...] + p

    @pl.when(i == NB - 1)
    def _():
        parts = []
        for v in (p1_ref[...], p2_ref[...], pool_acc[...]):
            parts.append(0.5 * (v[:, :HID] + v[:, HID:]))
        hcat = jnp.concatenate(parts, axis=1)
        t = jnp.maximum(
            jax.lax.dot_general(hcat, we1_ref[...], (((1,), (0,)), ((), ())),
                                preferred_element_type=jnp.float32)
            + be1_ref[...], 0.0)
        out_ref[...] = jax.lax.dot_general(
            t, we2_ref[...], (((1,), (0,)), ((), ())),
            preferred_element_type=jnp.float32) + be2_ref[...]


def _tc_last(h, agg, batch2d, w1, b1, w2, b2, p1, p2, we1, be1, we2, be2):
    return pl.pallas_call(
        _last_body,
        grid=(NB,),
        in_specs=[
            pl.BlockSpec((BN, 2 * HID), lambda i: (i, 0)),
            pl.BlockSpec((NCH, BN, DC), lambda i: (0, i, 0)),
            pl.BlockSpec((BN, 1), lambda i: (i, 0)),
            pl.BlockSpec(w1.shape, lambda i: (0, 0)),
            pl.BlockSpec(b1.shape, lambda i: (0, 0)),
            pl.BlockSpec(w2.shape, lambda i: (0, 0)),
            pl.BlockSpec(b2.shape, lambda i: (0, 0)),
            pl.BlockSpec(p1.shape, lambda i: (0, 0)),
            pl.BlockSpec(p2.shape, lambda i: (0, 0)),
            pl.BlockSpec(we1.shape, lambda i: (0, 0)),
            pl.BlockSpec(be1.shape, lambda i: (0, 0)),
            pl.BlockSpec(we2.shape, lambda i: (0, 0)),
            pl.BlockSpec(be2.shape, lambda i: (0, 0)),
        ],
        out_specs=pl.BlockSpec((G, we2.shape[1]), lambda i: (0, 0)),
        out_shape=jax.ShapeDtypeStruct((G, we2.shape[1]), jnp.float32),
        scratch_shapes=[pltpu.VMEM((G, 2 * HID), jnp.float32)],
    )(h, agg, batch2d, w1, b1, w2, b2, p1, p2, we1, be1, we2, be2)


def _pad_body(e_ref, src_ref, dst_ref):
    i = pl.program_id(0)
    nrows = e_ref.shape[1]
    row = lax.broadcasted_iota(jnp.int32, (nrows, K), 0) + i * nrows
    valid = row < (E // K)
    src_ref[...] = jnp.where(valid, e_ref[0], 0)
    dst_ref[...] = jnp.where(valid, e_ref[1], NP2)


def _pad_edges(edge2):
    nb = 8
    rows = EPAD // K // nb
    return pl.pallas_call(
        _pad_body,
        grid=(nb,),
        in_specs=[pl.BlockSpec((2, rows, K), lambda i: (0, i, 0))],
        out_specs=[
            pl.BlockSpec((rows, K), lambda i: (i, 0)),
            pl.BlockSpec((rows, K), lambda i: (i, 0)),
        ],
        out_shape=[
            jax.ShapeDtypeStruct((EPAD // K, K), jnp.int32),
            jax.ShapeDtypeStruct((EPAD // K, K), jnp.int32),
        ],
    )(edge2)


def _blockdiag(a, b):
    z = jnp.zeros((a.shape[0] + b.shape[0], a.shape[1] + b.shape[1]),
                  jnp.float32)
    z = z.at[:a.shape[0], :a.shape[1]].set(a)
    return z.at[a.shape[0]:, a.shape[1]:].set(b)


def kernel(pos, x, edge_index, batch, geo_params, topo_params, emb_params):
    f32 = jnp.float32
    edge2 = edge_index.astype(jnp.int32).reshape(2, E // K, K)
    src_p, dst_p = _pad_edges(edge2)
    feat0 = jnp.concatenate([pos.astype(f32), x[:, 3:8].astype(f32)], axis=1)
    batch2d = batch.astype(jnp.int32).reshape(N, 1)
    zeros0 = jnp.zeros((NPT, 8), f32)
    zeros1 = jnp.zeros((NPT, DC), f32)

    w1c, b1c, w2c, b2c = [], [], [], []
    for li in range(3):
        (w1g, b1g), (w2g, b2g) = geo_params[li]
        (w1t, b1t), (w2t, b2t) = topo_params[li]
        w1c.append(_blockdiag(w1g, w1t))
        b1c.append(jnp.concatenate([b1g, b1t]).reshape(1, 2 * HID))
        w2c.append(_blockdiag(w2g, w2t))
        b2c.append(jnp.concatenate([b2g, b2t]).reshape(1, 2 * HID))
    (we1, be1), (we2, be2) = emb_params
    be1 = be1.reshape(1, -1)
    be2 = be2.reshape(1, -1)

    agg0_fn = _make_agg(8, 1, True, 4, NI)
    agg_fn = _make_agg(DC, NCH, False, 2, NI2)

    agg0 = agg0_fn(feat0, src_p, dst_p, zeros0).reshape(2, NP2, 8)
    h1, p1 = _tc_layer(feat0, agg0, batch2d,
                       w1c[0], b1c[0], w2c[0], b2c[0], first=True)

    agg1 = agg_fn(h1.reshape(NCH * N, DC), src_p, dst_p,
                  zeros1).reshape(NCH, NP2, DC)
    h2, p2 = _tc_layer(h1, agg1, batch2d,
                       w1c[1], b1c[1], w2c[1], b2c[1], first=False)

    agg2 = agg_fn(h2.reshape(NCH * N, DC), src_p, dst_p,
                  zeros1).reshape(NCH, NP2, DC)
    out = _tc_last(h2, agg2, batch2d, w1c[2], b1c[2], w2c[2], b2c[2],
                   p1, p2, we1, be1, we2, be2)
    return out
